# Initial kernel scaffold; baseline (speedup 1.0000x reference)
import jax
import jax.numpy as jnp
from jax.experimental import pallas as pl


def kernel(x_user, x_question, x_answer, ei_asks, ei_answers, ei_contains, ei_rates, ei_similar, Wp_user, bp_user, Wp_question, bp_question, Wp_answer, bp_answer, mlp_W1, mlp_b1, mlp_g1, mlp_be1, mlp_W2, mlp_b2, mlp_g2, mlp_be2, epsilon, prelu_a, out_W1, out_b1, out_W2, out_b2):
    raise NotImplementedError("write your pallas kernel here")


# speedup vs baseline: 1.0421x; 1.0421x over previous
import functools

import jax
import jax.numpy as jnp
from jax import lax
from jax.experimental import pallas as pl
from jax.experimental.pallas import tpu as pltpu
from jax.experimental.pallas import tpu_sc as plsc

N = 50000
H = 128
E = 120000

BM = 400
GRID = N // BM
NT = 16
E2 = 122880
EPT = E2 // NT
NCH = 120
G = EPT // NCH
NPAD = 50176
RR = NPAD // 4
TRASH = 8
ZC = 28

_f32 = jnp.float32


def _sc_aggr_body(h_hbm, ed_hbm, aggr, acc, ed_v, idxm, gbuf):
    cid = lax.axis_index("c")
    tid = lax.axis_index("s")
    zrow0 = tid * (RR // NT)

    pltpu.sync_copy(ed_hbm.at[tid], ed_v)

    for p in range(2):
        base = (2 * cid + p) * RR

        @pl.loop(0, ZC)
        def _zfill(r):
            for k in range(8):
                gbuf[r, pl.ds(k * 16, 16)] = jnp.zeros((16,), _f32)

        @pl.loop(0, RR // NT // ZC)
        def _zero(z):
            pltpu.sync_copy(gbuf.at[pl.ds(0, ZC)],
                            acc.at[pl.ds(zrow0 + z * ZC, ZC)])

        plsc.subcore_barrier()

        @pl.loop(0, NCH)
        def _edges(j):
            @pl.loop(0, G // 16)
            def _remap(g):
                d = ed_v[j, pl.ds(G + g * 16, 16)]
                m = (d >= base) & (d < base + RR)
                t = RR + (lax.iota(jnp.int32, 16) & 7)
                idxm[0, pl.ds(g * 16, 16)] = jnp.where(m, d - base, t)

            pltpu.sync_copy(h_hbm.at[ed_v.at[j, pl.ds(0, G)]], gbuf)
            pltpu.sync_copy(gbuf, acc.at[idxm.at[0]], add=True)

        plsc.subcore_barrier()
        pltpu.sync_copy(acc.at[pl.ds(tid * (RR // NT), RR // NT)],
                        aggr.at[pl.ds(base + tid * (RR // NT), RR // NT)])
        plsc.subcore_barrier()


def _sc_aggr(h, ed):
    mesh = plsc.VectorSubcoreMesh(core_axis_name="c", subcore_axis_name="s",
                                  num_cores=2, num_subcores=NT)
    fn = pl.kernel(
        _sc_aggr_body,
        out_type=jax.ShapeDtypeStruct((NPAD, H), _f32),
        mesh=mesh,
        scratch_types=[
            pltpu.VMEM_SHARED((RR + TRASH, H), _f32),
            pltpu.VMEM((NCH, 2 * G), jnp.int32),
            pltpu.VMEM((1, G), jnp.int32),
            pltpu.VMEM((G, H), _f32),
        ],
    )
    return fn(h, ed)


_CP = pltpu.CompilerParams(dimension_semantics=("arbitrary",))


def _proj_body(x_ref, w_ref, b_ref, o_ref):
    o_ref[...] = (jnp.dot(x_ref[...], w_ref[...], preferred_element_type=_f32)
                  + b_ref[0:1, :])


def _proj(x, wpt, bp):
    return pl.pallas_call(
        _proj_body,
        grid=(GRID,),
        in_specs=[
            pl.BlockSpec((BM, H), lambda i: (i, 0)),
            pl.BlockSpec((H, H), lambda i: (0, 0)),
            pl.BlockSpec((8, H), lambda i: (0, 0)),
        ],
        out_specs=pl.BlockSpec((BM, H), lambda i: (i, 0)),
        out_shape=jax.ShapeDtypeStruct((NPAD, H), _f32),
        compiler_params=_CP,
    )(x, wpt, bp)


def _p1_body(e_ref, h_ref, a_ref, w_ref, u_ref, st_ref):
    i = pl.program_id(0)
    comb = e_ref[0, 0] * h_ref[...] + a_ref[...]
    u = jnp.dot(comb, w_ref[...], preferred_element_type=_f32)
    u_ref[...] = u
    s = jnp.sum(u, axis=0, keepdims=True)
    q = jnp.sum(u * u, axis=0, keepdims=True)
    st = jnp.concatenate([s, q, jnp.zeros((6, H), _f32)], axis=0)

    @pl.when(i == 0)
    def _():
        st_ref[...] = st

    @pl.when(i > 0)
    def _():
        st_ref[...] = st_ref[...] + st


def _p1(one_eps, h, aggr, w1t):
    return pl.pallas_call(
        _p1_body,
        grid=(GRID,),
        in_specs=[
            pl.BlockSpec(memory_space=pltpu.SMEM),
            pl.BlockSpec((BM, H), lambda i: (i, 0)),
            pl.BlockSpec((BM, H), lambda i: (i, 0)),
            pl.BlockSpec((H, H), lambda i: (0, 0)),
        ],
        out_specs=[
            pl.BlockSpec((BM, H), lambda i: (i, 0)),
            pl.BlockSpec((8, H), lambda i: (0, 0)),
        ],
        out_shape=[
            jax.ShapeDtypeStruct((N, H), _f32),
            jax.ShapeDtypeStruct((8, H), _f32),
        ],
        compiler_params=_CP,
    )(one_eps, h, aggr, w1t)


def _p2_body(u_ref, sc_ref, sh_ref, w_ref, o_ref, st_ref):
    i = pl.program_id(0)
    r = jnp.maximum(u_ref[...] * sc_ref[0:1, :] + sh_ref[0:1, :], 0.0)
    u2 = jnp.dot(r, w_ref[...], preferred_element_type=_f32)
    o_ref[...] = u2
    s = jnp.sum(u2, axis=0, keepdims=True)
    q = jnp.sum(u2 * u2, axis=0, keepdims=True)
    st = jnp.concatenate([s, q, jnp.zeros((6, H), _f32)], axis=0)

    @pl.when(i == 0)
    def _():
        st_ref[...] = st

    @pl.when(i > 0)
    def _():
        st_ref[...] = st_ref[...] + st


def _p2(u1, scale1, shift1, w2t):
    return pl.pallas_call(
        _p2_body,
        grid=(GRID,),
        in_specs=[
            pl.BlockSpec((BM, H), lambda i: (i, 0)),
            pl.BlockSpec((8, H), lambda i: (0, 0)),
            pl.BlockSpec((8, H), lambda i: (0, 0)),
            pl.BlockSpec((H, H), lambda i: (0, 0)),
        ],
        out_specs=[
            pl.BlockSpec((BM, H), lambda i: (i, 0)),
            pl.BlockSpec((8, H), lambda i: (0, 0)),
        ],
        out_shape=[
            jax.ShapeDtypeStruct((N, H), _f32),
            jax.ShapeDtypeStruct((8, H), _f32),
        ],
        compiler_params=_CP,
    )(u1, scale1, shift1, w2t)


def _combine_body(pa_ref, h_ref, v_ref, sc_ref, sh_ref, o_ref, *, prelu):
    x = h_ref[...] + v_ref[...] * sc_ref[0:1, :] + sh_ref[0:1, :]
    if prelu:
        x = jnp.where(x > 0, x, pa_ref[0, 0] * x)
    o_ref[...] = x


def _combine(pa, h, u2, scale2, shift2, prelu):
    return pl.pallas_call(
        functools.partial(_combine_body, prelu=prelu),
        grid=(GRID,),
        in_specs=[pl.BlockSpec(memory_space=pltpu.SMEM)]
        + [pl.BlockSpec((BM, H), lambda i: (i, 0)) for _ in range(2)]
        + [pl.BlockSpec((8, H), lambda i: (0, 0)) for _ in range(2)],
        out_specs=pl.BlockSpec((BM, H), lambda i: (i, 0)),
        out_shape=jax.ShapeDtypeStruct((NPAD, H), _f32),
        compiler_params=_CP,
    )(pa, h, u2, scale2, shift2)


def _out_body(h_ref, w1_ref, b1_ref, w2_ref, b2_ref, o_ref):
    t = jnp.dot(h_ref[...], w1_ref[...], preferred_element_type=_f32)
    t = jnp.maximum(t + b1_ref[0:1, :], 0.0)
    o_ref[...] = jnp.dot(t, w2_ref[...], preferred_element_type=_f32) + b2_ref[0:1, :]


def _outmlp(h, o1t, ob1, o2t, ob2):
    return pl.pallas_call(
        _out_body,
        grid=(GRID,),
        in_specs=[
            pl.BlockSpec((BM, H), lambda i: (i, 0)),
            pl.BlockSpec((H, H), lambda i: (0, 0)),
            pl.BlockSpec((8, H), lambda i: (0, 0)),
            pl.BlockSpec((H, H), lambda i: (0, 0)),
            pl.BlockSpec((8, H), lambda i: (0, 0)),
        ],
        out_specs=pl.BlockSpec((BM, H), lambda i: (i, 0)),
        out_shape=jax.ShapeDtypeStruct((N, H), _f32),
        compiler_params=_CP,
    )(h, o1t, ob1, o2t, ob2)


def _bn_scale_shift(st, g, be):
    s, q = st[0], st[1]
    m = s / N
    v = q / N - m * m
    scale = g * lax.rsqrt(v + 1e-5)
    shift = be - m * scale
    return (jnp.broadcast_to(scale[None, :], (8, H)),
            jnp.broadcast_to(shift[None, :], (8, H)))


def kernel(x_user, x_question, x_answer, ei_asks, ei_answers, ei_contains,
           ei_rates, ei_similar, Wp_user, bp_user, Wp_question, bp_question,
           Wp_answer, bp_answer, mlp_W1, mlp_b1, mlp_g1, mlp_be1, mlp_W2,
           mlp_b2, mlp_g2, mlp_be2, epsilon, prelu_a, out_W1, out_b1,
           out_W2, out_b2):
    one_eps = (1.0 + epsilon).astype(_f32).reshape(1, 1)
    pa = prelu_a.astype(_f32).reshape(1, 1)

    wpt = Wp_user.T
    bp = jnp.broadcast_to(bp_user[None, :], (8, H))
    w1t = mlp_W1[4].T
    w2t = mlp_W2[4].T
    g1, be1 = mlp_g1[4], mlp_be1[4]
    g2, be2 = mlp_g2[4], mlp_be2[4]
    o1t = out_W1.T
    ob1 = jnp.broadcast_to(out_b1[None, :], (8, H))
    o2t = out_W2.T
    ob2 = jnp.broadcast_to(out_b2[None, :], (8, H))

    pad = E2 - E
    src_p = jnp.concatenate([ei_similar[0], jnp.arange(pad, dtype=jnp.int32)])
    dst_p = jnp.concatenate(
        [ei_similar[1], jnp.full((pad,), NPAD, jnp.int32)])
    ed = jnp.concatenate([src_p.reshape(NT, NCH, G),
                          dst_p.reshape(NT, NCH, G)], axis=-1)

    h = _proj(x_user, wpt, bp)

    for layer in range(2):
        aggr = _sc_aggr(h, ed)
        u1, st1 = _p1(one_eps, h, aggr, w1t)
        scale1, shift1 = _bn_scale_shift(st1, g1, be1)
        u2, st2 = _p2(u1, scale1, shift1, w2t)
        scale2, shift2 = _bn_scale_shift(st2, g2, be2)
        h = _combine(pa, h, u2, scale2, shift2, prelu=(layer == 0))

    return _outmlp(h, o1t, ob1, o2t, ob2)


# speedup vs baseline: 1.3262x; 1.2726x over previous
import functools

import jax
import jax.numpy as jnp
from jax import lax
from jax.experimental import pallas as pl
from jax.experimental.pallas import tpu as pltpu
from jax.experimental.pallas import tpu_sc as plsc

N = 50000
H = 128
E = 120000

BM = 400
GRID = N // BM
NT = 16
E2 = 122880
EPT = E2 // NT
NCH = 120
G = EPT // NCH
NPAD = 50176
RR = NPAD // 4
TRASH = 8
ZC = 28

_f32 = jnp.float32


HCH = NCH // 3


def _sc_aggr_body(h_hbm, ed_hbm, aggr, acc, ed_v, idxm, ga, gb,
                  sga, sgb, ssa, ssb):
    cid = lax.axis_index("c")
    tid = lax.axis_index("s")
    zrow0 = tid * (RR // NT)

    def _wait(sem, buf):
        pltpu.make_async_copy(h_hbm.at[pl.ds(0, G)], buf, sem).wait()

    def _remap(j, b, base):
        @pl.loop(0, G // 16)
        def _g(g):
            d = ed_v[j, pl.ds(G + g * 16, 16)]
            m = (d >= base) & (d < base + RR)
            t = RR + (lax.iota(jnp.int32, 16) & 7)
            idxm[b, pl.ds(g * 16, 16)] = jnp.where(m, d - base, t)

    for p in range(2):
        base = (2 * cid + p) * RR

        @pl.loop(0, ZC)
        def _zfill(r):
            for k in range(8):
                ga[r, pl.ds(k * 16, 16)] = jnp.zeros((16,), _f32)

        @pl.loop(0, RR // NT // ZC)
        def _zero(z):
            pltpu.sync_copy(ga.at[pl.ds(0, ZC)],
                            acc.at[pl.ds(zrow0 + z * ZC, ZC)])

        plsc.subcore_barrier()

        for half in range(3):
            pltpu.sync_copy(ed_hbm.at[tid, pl.ds(half * HCH, HCH)], ed_v)
            pltpu.async_copy(h_hbm.at[ed_v.at[0, pl.ds(0, G)]], ga, sga)

            @pl.loop(0, HCH // 2)
            def _pair(q):
                j0 = 2 * q
                j1 = 2 * q + 1
                _remap(j0, 0, base)

                @pl.when(q > 0)
                def _():
                    _wait(ssb, gb)

                pltpu.async_copy(h_hbm.at[ed_v.at[j1, pl.ds(0, G)]], gb, sgb)
                _wait(sga, ga)
                pltpu.async_copy(ga, acc.at[idxm.at[0]], ssa, add=True)
                _remap(j1, 1, base)
                _wait(ssa, ga)

                @pl.when(q < HCH // 2 - 1)
                def _():
                    pltpu.async_copy(
                        h_hbm.at[ed_v.at[j0 + 2, pl.ds(0, G)]], ga, sga)

                _wait(sgb, gb)
                pltpu.async_copy(gb, acc.at[idxm.at[1]], ssb, add=True)

            _wait(ssb, gb)

        plsc.subcore_barrier()
        pltpu.sync_copy(acc.at[pl.ds(tid * (RR // NT), RR // NT)],
                        aggr.at[pl.ds(base + tid * (RR // NT), RR // NT)])
        plsc.subcore_barrier()


def _sc_aggr(h, ed):
    mesh = plsc.VectorSubcoreMesh(core_axis_name="c", subcore_axis_name="s",
                                  num_cores=2, num_subcores=NT)
    fn = pl.kernel(
        _sc_aggr_body,
        out_type=jax.ShapeDtypeStruct((NPAD, H), _f32),
        mesh=mesh,
        scratch_types=[
            pltpu.VMEM_SHARED((RR + TRASH, H), _f32),
            pltpu.VMEM((HCH, 2 * G), jnp.int32),
            pltpu.VMEM((2, G), jnp.int32),
            pltpu.VMEM((G, H), _f32),
            pltpu.VMEM((G, H), _f32),
            pltpu.SemaphoreType.DMA,
            pltpu.SemaphoreType.DMA,
            pltpu.SemaphoreType.DMA,
            pltpu.SemaphoreType.DMA,
        ],
    )
    return fn(h, ed)


_CP = pltpu.CompilerParams(dimension_semantics=("arbitrary",))


def _proj_body(x_ref, w_ref, b_ref, o_ref):
    o_ref[...] = (jnp.dot(x_ref[...], w_ref[...], preferred_element_type=_f32)
                  + b_ref[0:1, :])


def _proj(x, wpt, bp):
    return pl.pallas_call(
        _proj_body,
        grid=(GRID,),
        in_specs=[
            pl.BlockSpec((BM, H), lambda i: (i, 0)),
            pl.BlockSpec((H, H), lambda i: (0, 0)),
            pl.BlockSpec((8, H), lambda i: (0, 0)),
        ],
        out_specs=pl.BlockSpec((BM, H), lambda i: (i, 0)),
        out_shape=jax.ShapeDtypeStruct((NPAD, H), _f32),
        compiler_params=_CP,
    )(x, wpt, bp)


def _p1_body(e_ref, h_ref, a_ref, w_ref, u_ref, st_ref):
    i = pl.program_id(0)
    comb = e_ref[0, 0] * h_ref[...] + a_ref[...]
    u = jnp.dot(comb, w_ref[...], preferred_element_type=_f32)
    u_ref[...] = u
    s = jnp.sum(u, axis=0, keepdims=True)
    q = jnp.sum(u * u, axis=0, keepdims=True)
    st = jnp.concatenate([s, q, jnp.zeros((6, H), _f32)], axis=0)

    @pl.when(i == 0)
    def _():
        st_ref[...] = st

    @pl.when(i > 0)
    def _():
        st_ref[...] = st_ref[...] + st


def _p1(one_eps, h, aggr, w1t):
    return pl.pallas_call(
        _p1_body,
        grid=(GRID,),
        in_specs=[
            pl.BlockSpec(memory_space=pltpu.SMEM),
            pl.BlockSpec((BM, H), lambda i: (i, 0)),
            pl.BlockSpec((BM, H), lambda i: (i, 0)),
            pl.BlockSpec((H, H), lambda i: (0, 0)),
        ],
        out_specs=[
            pl.BlockSpec((BM, H), lambda i: (i, 0)),
            pl.BlockSpec((8, H), lambda i: (0, 0)),
        ],
        out_shape=[
            jax.ShapeDtypeStruct((N, H), _f32),
            jax.ShapeDtypeStruct((8, H), _f32),
        ],
        compiler_params=_CP,
    )(one_eps, h, aggr, w1t)


def _p2_body(u_ref, sc_ref, sh_ref, w_ref, o_ref, st_ref):
    i = pl.program_id(0)
    r = jnp.maximum(u_ref[...] * sc_ref[0:1, :] + sh_ref[0:1, :], 0.0)
    u2 = jnp.dot(r, w_ref[...], preferred_element_type=_f32)
    o_ref[...] = u2
    s = jnp.sum(u2, axis=0, keepdims=True)
    q = jnp.sum(u2 * u2, axis=0, keepdims=True)
    st = jnp.concatenate([s, q, jnp.zeros((6, H), _f32)], axis=0)

    @pl.when(i == 0)
    def _():
        st_ref[...] = st

    @pl.when(i > 0)
    def _():
        st_ref[...] = st_ref[...] + st


def _p2(u1, scale1, shift1, w2t):
    return pl.pallas_call(
        _p2_body,
        grid=(GRID,),
        in_specs=[
            pl.BlockSpec((BM, H), lambda i: (i, 0)),
            pl.BlockSpec((8, H), lambda i: (0, 0)),
            pl.BlockSpec((8, H), lambda i: (0, 0)),
            pl.BlockSpec((H, H), lambda i: (0, 0)),
        ],
        out_specs=[
            pl.BlockSpec((BM, H), lambda i: (i, 0)),
            pl.BlockSpec((8, H), lambda i: (0, 0)),
        ],
        out_shape=[
            jax.ShapeDtypeStruct((N, H), _f32),
            jax.ShapeDtypeStruct((8, H), _f32),
        ],
        compiler_params=_CP,
    )(u1, scale1, shift1, w2t)


def _combine_body(pa_ref, h_ref, v_ref, sc_ref, sh_ref, o_ref, *, prelu):
    x = h_ref[...] + v_ref[...] * sc_ref[0:1, :] + sh_ref[0:1, :]
    if prelu:
        x = jnp.where(x > 0, x, pa_ref[0, 0] * x)
    o_ref[...] = x


def _combine(pa, h, u2, scale2, shift2, prelu):
    return pl.pallas_call(
        functools.partial(_combine_body, prelu=prelu),
        grid=(GRID,),
        in_specs=[pl.BlockSpec(memory_space=pltpu.SMEM)]
        + [pl.BlockSpec((BM, H), lambda i: (i, 0)) for _ in range(2)]
        + [pl.BlockSpec((8, H), lambda i: (0, 0)) for _ in range(2)],
        out_specs=pl.BlockSpec((BM, H), lambda i: (i, 0)),
        out_shape=jax.ShapeDtypeStruct((NPAD, H), _f32),
        compiler_params=_CP,
    )(pa, h, u2, scale2, shift2)


def _out_body(h_ref, w1_ref, b1_ref, w2_ref, b2_ref, o_ref):
    t = jnp.dot(h_ref[...], w1_ref[...], preferred_element_type=_f32)
    t = jnp.maximum(t + b1_ref[0:1, :], 0.0)
    o_ref[...] = jnp.dot(t, w2_ref[...], preferred_element_type=_f32) + b2_ref[0:1, :]


def _outmlp(h, o1t, ob1, o2t, ob2):
    return pl.pallas_call(
        _out_body,
        grid=(GRID,),
        in_specs=[
            pl.BlockSpec((BM, H), lambda i: (i, 0)),
            pl.BlockSpec((H, H), lambda i: (0, 0)),
            pl.BlockSpec((8, H), lambda i: (0, 0)),
            pl.BlockSpec((H, H), lambda i: (0, 0)),
            pl.BlockSpec((8, H), lambda i: (0, 0)),
        ],
        out_specs=pl.BlockSpec((BM, H), lambda i: (i, 0)),
        out_shape=jax.ShapeDtypeStruct((N, H), _f32),
        compiler_params=_CP,
    )(h, o1t, ob1, o2t, ob2)


def _bn_scale_shift(st, g, be):
    s, q = st[0], st[1]
    m = s / N
    v = q / N - m * m
    scale = g * lax.rsqrt(v + 1e-5)
    shift = be - m * scale
    return (jnp.broadcast_to(scale[None, :], (8, H)),
            jnp.broadcast_to(shift[None, :], (8, H)))


def kernel(x_user, x_question, x_answer, ei_asks, ei_answers, ei_contains,
           ei_rates, ei_similar, Wp_user, bp_user, Wp_question, bp_question,
           Wp_answer, bp_answer, mlp_W1, mlp_b1, mlp_g1, mlp_be1, mlp_W2,
           mlp_b2, mlp_g2, mlp_be2, epsilon, prelu_a, out_W1, out_b1,
           out_W2, out_b2):
    one_eps = (1.0 + epsilon).astype(_f32).reshape(1, 1)
    pa = prelu_a.astype(_f32).reshape(1, 1)

    wpt = Wp_user.T
    bp = jnp.broadcast_to(bp_user[None, :], (8, H))
    w1t = mlp_W1[4].T
    w2t = mlp_W2[4].T
    g1, be1 = mlp_g1[4], mlp_be1[4]
    g2, be2 = mlp_g2[4], mlp_be2[4]
    o1t = out_W1.T
    ob1 = jnp.broadcast_to(out_b1[None, :], (8, H))
    o2t = out_W2.T
    ob2 = jnp.broadcast_to(out_b2[None, :], (8, H))

    pad = E2 - E
    src_p = jnp.concatenate([ei_similar[0], jnp.arange(pad, dtype=jnp.int32)])
    dst_p = jnp.concatenate(
        [ei_similar[1], jnp.full((pad,), NPAD, jnp.int32)])
    ed = jnp.concatenate([src_p.reshape(NT, NCH, G),
                          dst_p.reshape(NT, NCH, G)], axis=-1)

    h = _proj(x_user, wpt, bp)

    for layer in range(2):
        aggr = _sc_aggr(h, ed)
        u1, st1 = _p1(one_eps, h, aggr, w1t)
        scale1, shift1 = _bn_scale_shift(st1, g1, be1)
        u2, st2 = _p2(u1, scale1, shift1, w2t)
        scale2, shift2 = _bn_scale_shift(st2, g2, be2)
        h = _combine(pa, h, u2, scale2, shift2, prelu=(layer == 0))

    return _outmlp(h, o1t, ob1, o2t, ob2)


# speedup vs baseline: 1.3279x; 1.0013x over previous
import functools

import jax
import jax.numpy as jnp
from jax import lax
from jax.experimental import pallas as pl
from jax.experimental.pallas import tpu as pltpu
from jax.experimental.pallas import tpu_sc as plsc

N = 50000
H = 128
E = 120000

BM = 400
GRID = N // BM
NT = 16
E2 = 122880
EPT = E2 // NT
NCH = 120
G = EPT // NCH
NPAD = 50176
RR = NPAD // 4
TRASH = 256
ZC = 28

_f32 = jnp.float32


HCH = NCH // 3


def _sc_aggr_body(h_hbm, ed_hbm, aggr, acc, ed_v, idxm, ga, gb,
                  sga, sgb, ssa, ssb):
    cid = lax.axis_index("c")
    tid = lax.axis_index("s")
    zrow0 = tid * (RR // NT)

    def _wait(sem, buf):
        pltpu.make_async_copy(h_hbm.at[pl.ds(0, G)], buf, sem).wait()

    def _remap(j, b, base):
        @pl.loop(0, G // 16)
        def _g(g):
            d = ed_v[j, pl.ds(G + g * 16, 16)]
            m = (d >= base) & (d < base + RR)
            t = RR + ((j * G + g * 16 + lax.iota(jnp.int32, 16)) & (TRASH - 1))
            idxm[b, pl.ds(g * 16, 16)] = jnp.where(m, d - base, t)

    for p in range(2):
        base = (2 * cid + p) * RR

        @pl.loop(0, ZC)
        def _zfill(r):
            for k in range(8):
                ga[r, pl.ds(k * 16, 16)] = jnp.zeros((16,), _f32)

        @pl.loop(0, RR // NT // ZC)
        def _zero(z):
            pltpu.sync_copy(ga.at[pl.ds(0, ZC)],
                            acc.at[pl.ds(zrow0 + z * ZC, ZC)])

        plsc.subcore_barrier()

        for half in range(3):
            pltpu.sync_copy(ed_hbm.at[tid, pl.ds(half * HCH, HCH)], ed_v)
            pltpu.async_copy(h_hbm.at[ed_v.at[0, pl.ds(0, G)]], ga, sga)

            @pl.loop(0, HCH // 2)
            def _pair(q):
                j0 = 2 * q
                j1 = 2 * q + 1
                _remap(j0, 0, base)

                @pl.when(q > 0)
                def _():
                    _wait(ssb, gb)

                pltpu.async_copy(h_hbm.at[ed_v.at[j1, pl.ds(0, G)]], gb, sgb)
                _wait(sga, ga)
                pltpu.async_copy(ga, acc.at[idxm.at[0]], ssa, add=True)
                _remap(j1, 1, base)
                _wait(ssa, ga)

                @pl.when(q < HCH // 2 - 1)
                def _():
                    pltpu.async_copy(
                        h_hbm.at[ed_v.at[j0 + 2, pl.ds(0, G)]], ga, sga)

                _wait(sgb, gb)
                pltpu.async_copy(gb, acc.at[idxm.at[1]], ssb, add=True)

            _wait(ssb, gb)

        plsc.subcore_barrier()
        pltpu.sync_copy(acc.at[pl.ds(tid * (RR // NT), RR // NT)],
                        aggr.at[pl.ds(base + tid * (RR // NT), RR // NT)])
        plsc.subcore_barrier()


def _sc_aggr(h, ed):
    mesh = plsc.VectorSubcoreMesh(core_axis_name="c", subcore_axis_name="s",
                                  num_cores=2, num_subcores=NT)
    fn = pl.kernel(
        _sc_aggr_body,
        out_type=jax.ShapeDtypeStruct((NPAD, H), _f32),
        mesh=mesh,
        scratch_types=[
            pltpu.VMEM_SHARED((RR + TRASH, H), _f32),
            pltpu.VMEM((HCH, 2 * G), jnp.int32),
            pltpu.VMEM((2, G), jnp.int32),
            pltpu.VMEM((G, H), _f32),
            pltpu.VMEM((G, H), _f32),
            pltpu.SemaphoreType.DMA,
            pltpu.SemaphoreType.DMA,
            pltpu.SemaphoreType.DMA,
            pltpu.SemaphoreType.DMA,
        ],
    )
    return fn(h, ed)


_CP = pltpu.CompilerParams(dimension_semantics=("arbitrary",))


def _proj_body(x_ref, w_ref, b_ref, o_ref):
    o_ref[...] = (jnp.dot(x_ref[...], w_ref[...], preferred_element_type=_f32)
                  + b_ref[0:1, :])


def _proj(x, wpt, bp):
    return pl.pallas_call(
        _proj_body,
        grid=(GRID,),
        in_specs=[
            pl.BlockSpec((BM, H), lambda i: (i, 0)),
            pl.BlockSpec((H, H), lambda i: (0, 0)),
            pl.BlockSpec((8, H), lambda i: (0, 0)),
        ],
        out_specs=pl.BlockSpec((BM, H), lambda i: (i, 0)),
        out_shape=jax.ShapeDtypeStruct((NPAD, H), _f32),
        compiler_params=_CP,
    )(x, wpt, bp)


def _p1_body(e_ref, h_ref, a_ref, w_ref, u_ref, st_ref):
    i = pl.program_id(0)
    comb = e_ref[0, 0] * h_ref[...] + a_ref[...]
    u = jnp.dot(comb, w_ref[...], preferred_element_type=_f32)
    u_ref[...] = u
    s = jnp.sum(u, axis=0, keepdims=True)
    q = jnp.sum(u * u, axis=0, keepdims=True)
    st = jnp.concatenate([s, q, jnp.zeros((6, H), _f32)], axis=0)

    @pl.when(i == 0)
    def _():
        st_ref[...] = st

    @pl.when(i > 0)
    def _():
        st_ref[...] = st_ref[...] + st


def _p1(one_eps, h, aggr, w1t):
    return pl.pallas_call(
        _p1_body,
        grid=(GRID,),
        in_specs=[
            pl.BlockSpec(memory_space=pltpu.SMEM),
            pl.BlockSpec((BM, H), lambda i: (i, 0)),
            pl.BlockSpec((BM, H), lambda i: (i, 0)),
            pl.BlockSpec((H, H), lambda i: (0, 0)),
        ],
        out_specs=[
            pl.BlockSpec((BM, H), lambda i: (i, 0)),
            pl.BlockSpec((8, H), lambda i: (0, 0)),
        ],
        out_shape=[
            jax.ShapeDtypeStruct((N, H), _f32),
            jax.ShapeDtypeStruct((8, H), _f32),
        ],
        compiler_params=_CP,
    )(one_eps, h, aggr, w1t)


def _p2_body(u_ref, st1_ref, g_ref, be_ref, w_ref, o_ref, st_ref):
    i = pl.program_id(0)
    m = st1_ref[0:1, :] * (1.0 / N)
    v = st1_ref[1:2, :] * (1.0 / N) - m * m
    scale = g_ref[0:1, :] * lax.rsqrt(v + 1e-5)
    shift = be_ref[0:1, :] - m * scale
    r = jnp.maximum(u_ref[...] * scale + shift, 0.0)
    u2 = jnp.dot(r, w_ref[...], preferred_element_type=_f32)
    o_ref[...] = u2
    s = jnp.sum(u2, axis=0, keepdims=True)
    q = jnp.sum(u2 * u2, axis=0, keepdims=True)
    st = jnp.concatenate([s, q, jnp.zeros((6, H), _f32)], axis=0)

    @pl.when(i == 0)
    def _():
        st_ref[...] = st

    @pl.when(i > 0)
    def _():
        st_ref[...] = st_ref[...] + st


def _p2(u1, st1, g1b, be1b, w2t):
    return pl.pallas_call(
        _p2_body,
        grid=(GRID,),
        in_specs=[
            pl.BlockSpec((BM, H), lambda i: (i, 0)),
            pl.BlockSpec((8, H), lambda i: (0, 0)),
            pl.BlockSpec((8, H), lambda i: (0, 0)),
            pl.BlockSpec((8, H), lambda i: (0, 0)),
            pl.BlockSpec((H, H), lambda i: (0, 0)),
        ],
        out_specs=[
            pl.BlockSpec((BM, H), lambda i: (i, 0)),
            pl.BlockSpec((8, H), lambda i: (0, 0)),
        ],
        out_shape=[
            jax.ShapeDtypeStruct((N, H), _f32),
            jax.ShapeDtypeStruct((8, H), _f32),
        ],
        compiler_params=_CP,
    )(u1, st1, g1b, be1b, w2t)


def _combine_body(pa_ref, h_ref, v_ref, st2_ref, g_ref, be_ref, o_ref, *,
                  prelu):
    m = st2_ref[0:1, :] * (1.0 / N)
    v = st2_ref[1:2, :] * (1.0 / N) - m * m
    scale = g_ref[0:1, :] * lax.rsqrt(v + 1e-5)
    shift = be_ref[0:1, :] - m * scale
    x = h_ref[...] + v_ref[...] * scale + shift
    if prelu:
        x = jnp.where(x > 0, x, pa_ref[0, 0] * x)
    o_ref[...] = x


def _combine(pa, h, u2, st2, g2b, be2b, prelu):
    return pl.pallas_call(
        functools.partial(_combine_body, prelu=prelu),
        grid=(GRID,),
        in_specs=[pl.BlockSpec(memory_space=pltpu.SMEM)]
        + [pl.BlockSpec((BM, H), lambda i: (i, 0)) for _ in range(2)]
        + [pl.BlockSpec((8, H), lambda i: (0, 0)) for _ in range(3)],
        out_specs=pl.BlockSpec((BM, H), lambda i: (i, 0)),
        out_shape=jax.ShapeDtypeStruct((NPAD, H), _f32),
        compiler_params=_CP,
    )(pa, h, u2, st2, g2b, be2b)


def _out_body(h_ref, w1_ref, b1_ref, w2_ref, b2_ref, o_ref):
    t = jnp.dot(h_ref[...], w1_ref[...], preferred_element_type=_f32)
    t = jnp.maximum(t + b1_ref[0:1, :], 0.0)
    o_ref[...] = jnp.dot(t, w2_ref[...], preferred_element_type=_f32) + b2_ref[0:1, :]


def _outmlp(h, o1t, ob1, o2t, ob2):
    return pl.pallas_call(
        _out_body,
        grid=(GRID,),
        in_specs=[
            pl.BlockSpec((BM, H), lambda i: (i, 0)),
            pl.BlockSpec((H, H), lambda i: (0, 0)),
            pl.BlockSpec((8, H), lambda i: (0, 0)),
            pl.BlockSpec((H, H), lambda i: (0, 0)),
            pl.BlockSpec((8, H), lambda i: (0, 0)),
        ],
        out_specs=pl.BlockSpec((BM, H), lambda i: (i, 0)),
        out_shape=jax.ShapeDtypeStruct((N, H), _f32),
        compiler_params=_CP,
    )(h, o1t, ob1, o2t, ob2)


def kernel(x_user, x_question, x_answer, ei_asks, ei_answers, ei_contains,
           ei_rates, ei_similar, Wp_user, bp_user, Wp_question, bp_question,
           Wp_answer, bp_answer, mlp_W1, mlp_b1, mlp_g1, mlp_be1, mlp_W2,
           mlp_b2, mlp_g2, mlp_be2, epsilon, prelu_a, out_W1, out_b1,
           out_W2, out_b2):
    one_eps = (1.0 + epsilon).astype(_f32).reshape(1, 1)
    pa = prelu_a.astype(_f32).reshape(1, 1)

    wpt = Wp_user.T
    bp = jnp.broadcast_to(bp_user[None, :], (8, H))
    w1t = mlp_W1[4].T
    w2t = mlp_W2[4].T
    g1b = jnp.broadcast_to(mlp_g1[4][None, :], (8, H))
    be1b = jnp.broadcast_to(mlp_be1[4][None, :], (8, H))
    g2b = jnp.broadcast_to(mlp_g2[4][None, :], (8, H))
    be2b = jnp.broadcast_to(mlp_be2[4][None, :], (8, H))
    o1t = out_W1.T
    ob1 = jnp.broadcast_to(out_b1[None, :], (8, H))
    o2t = out_W2.T
    ob2 = jnp.broadcast_to(out_b2[None, :], (8, H))

    pad = E2 - E
    src_p = jnp.concatenate([ei_similar[0], jnp.arange(pad, dtype=jnp.int32)])
    dst_p = jnp.concatenate(
        [ei_similar[1], jnp.full((pad,), NPAD, jnp.int32)])
    ed = jnp.concatenate([src_p.reshape(NT, NCH, G),
                          dst_p.reshape(NT, NCH, G)], axis=-1)

    h = _proj(x_user, wpt, bp)

    for layer in range(2):
        aggr = _sc_aggr(h, ed)
        u1, st1 = _p1(one_eps, h, aggr, w1t)
        u2, st2 = _p2(u1, st1, g1b, be1b, w2t)
        h = _combine(pa, h, u2, st2, g2b, be2b, prelu=(layer == 0))

    return _outmlp(h, o1t, ob1, o2t, ob2)


# speedup vs baseline: 2.1163x; 1.5937x over previous
import functools

import jax
import jax.numpy as jnp
from jax import lax
from jax.experimental import pallas as pl
from jax.experimental.pallas import tpu as pltpu
from jax.experimental.pallas import tpu_sc as plsc

N = 50000
H = 128
E = 120000

BM = 2000
GRID = N // BM
NT = 16
E2 = 122880
EPT = E2 // NT
NCH = 120
G = EPT // NCH
NPAD = 50176
RR = NPAD // 4
TRASH = 256
ZC = 28

_f32 = jnp.float32


HCH = NCH // 3


def _sc_aggr_body(h_hbm, ed_hbm, aggr, acc, ed_v, idxm, ga, gb,
                  sga, sgb, ssa, ssb):
    cid = lax.axis_index("c")
    tid = lax.axis_index("s")
    zrow0 = tid * (RR // NT)

    def _wait(sem, buf):
        pltpu.make_async_copy(h_hbm.at[pl.ds(0, G)], buf, sem).wait()

    def _remap(j, b, base):
        @pl.loop(0, G // 16)
        def _g(g):
            d = ed_v[j, pl.ds(G + g * 16, 16)]
            m = (d >= base) & (d < base + RR)
            t = RR + ((j * G + g * 16 + lax.iota(jnp.int32, 16)) & (TRASH - 1))
            idxm[b, pl.ds(g * 16, 16)] = jnp.where(m, d - base, t)

    for p in range(2):
        base = (2 * cid + p) * RR

        @pl.loop(0, ZC)
        def _zfill(r):
            for k in range(8):
                ga[r, pl.ds(k * 16, 16)] = jnp.zeros((16,), _f32)

        @pl.loop(0, RR // NT // ZC)
        def _zero(z):
            pltpu.sync_copy(ga.at[pl.ds(0, ZC)],
                            acc.at[pl.ds(zrow0 + z * ZC, ZC)])

        plsc.subcore_barrier()

        for half in range(3):
            pltpu.sync_copy(ed_hbm.at[tid, pl.ds(half * HCH, HCH)], ed_v)
            pltpu.async_copy(h_hbm.at[ed_v.at[0, pl.ds(0, G)]], ga, sga)

            @pl.loop(0, HCH // 2)
            def _pair(q):
                j0 = 2 * q
                j1 = 2 * q + 1
                _remap(j0, 0, base)

                @pl.when(q > 0)
                def _():
                    _wait(ssb, gb)

                pltpu.async_copy(h_hbm.at[ed_v.at[j1, pl.ds(0, G)]], gb, sgb)
                _wait(sga, ga)
                pltpu.async_copy(ga, acc.at[idxm.at[0]], ssa, add=True)
                _remap(j1, 1, base)
                _wait(ssa, ga)

                @pl.when(q < HCH // 2 - 1)
                def _():
                    pltpu.async_copy(
                        h_hbm.at[ed_v.at[j0 + 2, pl.ds(0, G)]], ga, sga)

                _wait(sgb, gb)
                pltpu.async_copy(gb, acc.at[idxm.at[1]], ssb, add=True)

            _wait(ssb, gb)

        plsc.subcore_barrier()
        pltpu.sync_copy(acc.at[pl.ds(tid * (RR // NT), RR // NT)],
                        aggr.at[pl.ds(base + tid * (RR // NT), RR // NT)])
        plsc.subcore_barrier()


def _sc_aggr(h, ed):
    mesh = plsc.VectorSubcoreMesh(core_axis_name="c", subcore_axis_name="s",
                                  num_cores=2, num_subcores=NT)
    fn = pl.kernel(
        _sc_aggr_body,
        out_type=jax.ShapeDtypeStruct((NPAD, H), _f32),
        mesh=mesh,
        scratch_types=[
            pltpu.VMEM_SHARED((RR + TRASH, H), _f32),
            pltpu.VMEM((HCH, 2 * G), jnp.int32),
            pltpu.VMEM((2, G), jnp.int32),
            pltpu.VMEM((G, H), _f32),
            pltpu.VMEM((G, H), _f32),
            pltpu.SemaphoreType.DMA,
            pltpu.SemaphoreType.DMA,
            pltpu.SemaphoreType.DMA,
            pltpu.SemaphoreType.DMA,
        ],
    )
    return fn(h, ed)


_CP = pltpu.CompilerParams(dimension_semantics=("arbitrary",))


def _proj_body(x_ref, w_ref, b_ref, o_ref):
    o_ref[...] = (jnp.dot(x_ref[...], w_ref[...], preferred_element_type=_f32)
                  + b_ref[0:1, :])


def _proj(x, wpt, bp):
    return pl.pallas_call(
        _proj_body,
        grid=(GRID,),
        in_specs=[
            pl.BlockSpec((BM, H), lambda i: (i, 0)),
            pl.BlockSpec((H, H), lambda i: (0, 0)),
            pl.BlockSpec((8, H), lambda i: (0, 0)),
        ],
        out_specs=pl.BlockSpec((BM, H), lambda i: (i, 0)),
        out_shape=jax.ShapeDtypeStruct((NPAD, H), _f32),
        compiler_params=_CP,
    )(x, wpt, bp)


def _p1_body(e_ref, h_ref, a_ref, w_ref, u_ref, st_ref):
    i = pl.program_id(0)
    comb = e_ref[0, 0] * h_ref[...] + a_ref[...]
    u = jnp.dot(comb, w_ref[...], preferred_element_type=_f32)
    u_ref[...] = u
    s = jnp.sum(u, axis=0, keepdims=True)
    q = jnp.sum(u * u, axis=0, keepdims=True)
    st = jnp.concatenate([s, q, jnp.zeros((6, H), _f32)], axis=0)

    @pl.when(i == 0)
    def _():
        st_ref[...] = st

    @pl.when(i > 0)
    def _():
        st_ref[...] = st_ref[...] + st


def _p1(one_eps, h, aggr, w1t):
    return pl.pallas_call(
        _p1_body,
        grid=(GRID,),
        in_specs=[
            pl.BlockSpec(memory_space=pltpu.SMEM),
            pl.BlockSpec((BM, H), lambda i: (i, 0)),
            pl.BlockSpec((BM, H), lambda i: (i, 0)),
            pl.BlockSpec((H, H), lambda i: (0, 0)),
        ],
        out_specs=[
            pl.BlockSpec((BM, H), lambda i: (i, 0)),
            pl.BlockSpec((8, H), lambda i: (0, 0)),
        ],
        out_shape=[
            jax.ShapeDtypeStruct((N, H), _f32),
            jax.ShapeDtypeStruct((8, H), _f32),
        ],
        compiler_params=_CP,
    )(one_eps, h, aggr, w1t)


def _p2_body(u_ref, st1_ref, g_ref, be_ref, w_ref, o_ref, st_ref):
    i = pl.program_id(0)
    m = st1_ref[0:1, :] * (1.0 / N)
    v = st1_ref[1:2, :] * (1.0 / N) - m * m
    scale = g_ref[0:1, :] * lax.rsqrt(v + 1e-5)
    shift = be_ref[0:1, :] - m * scale
    r = jnp.maximum(u_ref[...] * scale + shift, 0.0)
    u2 = jnp.dot(r, w_ref[...], preferred_element_type=_f32)
    o_ref[...] = u2
    s = jnp.sum(u2, axis=0, keepdims=True)
    q = jnp.sum(u2 * u2, axis=0, keepdims=True)
    st = jnp.concatenate([s, q, jnp.zeros((6, H), _f32)], axis=0)

    @pl.when(i == 0)
    def _():
        st_ref[...] = st

    @pl.when(i > 0)
    def _():
        st_ref[...] = st_ref[...] + st


def _p2(u1, st1, g1b, be1b, w2t):
    return pl.pallas_call(
        _p2_body,
        grid=(GRID,),
        in_specs=[
            pl.BlockSpec((BM, H), lambda i: (i, 0)),
            pl.BlockSpec((8, H), lambda i: (0, 0)),
            pl.BlockSpec((8, H), lambda i: (0, 0)),
            pl.BlockSpec((8, H), lambda i: (0, 0)),
            pl.BlockSpec((H, H), lambda i: (0, 0)),
        ],
        out_specs=[
            pl.BlockSpec((BM, H), lambda i: (i, 0)),
            pl.BlockSpec((8, H), lambda i: (0, 0)),
        ],
        out_shape=[
            jax.ShapeDtypeStruct((N, H), _f32),
            jax.ShapeDtypeStruct((8, H), _f32),
        ],
        compiler_params=_CP,
    )(u1, st1, g1b, be1b, w2t)


def _combine_body(pa_ref, h_ref, v_ref, st2_ref, g_ref, be_ref, o_ref, *,
                  prelu):
    m = st2_ref[0:1, :] * (1.0 / N)
    v = st2_ref[1:2, :] * (1.0 / N) - m * m
    scale = g_ref[0:1, :] * lax.rsqrt(v + 1e-5)
    shift = be_ref[0:1, :] - m * scale
    x = h_ref[...] + v_ref[...] * scale + shift
    if prelu:
        x = jnp.where(x > 0, x, pa_ref[0, 0] * x)
    o_ref[...] = x


def _combine(pa, h, u2, st2, g2b, be2b, prelu):
    return pl.pallas_call(
        functools.partial(_combine_body, prelu=prelu),
        grid=(GRID,),
        in_specs=[pl.BlockSpec(memory_space=pltpu.SMEM)]
        + [pl.BlockSpec((BM, H), lambda i: (i, 0)) for _ in range(2)]
        + [pl.BlockSpec((8, H), lambda i: (0, 0)) for _ in range(3)],
        out_specs=pl.BlockSpec((BM, H), lambda i: (i, 0)),
        out_shape=jax.ShapeDtypeStruct((NPAD, H), _f32),
        compiler_params=_CP,
    )(pa, h, u2, st2, g2b, be2b)


def _out_body(h_ref, w1_ref, b1_ref, w2_ref, b2_ref, o_ref):
    t = jnp.dot(h_ref[...], w1_ref[...], preferred_element_type=_f32)
    t = jnp.maximum(t + b1_ref[0:1, :], 0.0)
    o_ref[...] = jnp.dot(t, w2_ref[...], preferred_element_type=_f32) + b2_ref[0:1, :]


def _outmlp(h, o1t, ob1, o2t, ob2):
    return pl.pallas_call(
        _out_body,
        grid=(GRID,),
        in_specs=[
            pl.BlockSpec((BM, H), lambda i: (i, 0)),
            pl.BlockSpec((H, H), lambda i: (0, 0)),
            pl.BlockSpec((8, H), lambda i: (0, 0)),
            pl.BlockSpec((H, H), lambda i: (0, 0)),
            pl.BlockSpec((8, H), lambda i: (0, 0)),
        ],
        out_specs=pl.BlockSpec((BM, H), lambda i: (i, 0)),
        out_shape=jax.ShapeDtypeStruct((N, H), _f32),
        compiler_params=_CP,
    )(h, o1t, ob1, o2t, ob2)


def kernel(x_user, x_question, x_answer, ei_asks, ei_answers, ei_contains,
           ei_rates, ei_similar, Wp_user, bp_user, Wp_question, bp_question,
           Wp_answer, bp_answer, mlp_W1, mlp_b1, mlp_g1, mlp_be1, mlp_W2,
           mlp_b2, mlp_g2, mlp_be2, epsilon, prelu_a, out_W1, out_b1,
           out_W2, out_b2):
    one_eps = (1.0 + epsilon).astype(_f32).reshape(1, 1)
    pa = prelu_a.astype(_f32).reshape(1, 1)

    wpt = Wp_user.T
    bp = jnp.broadcast_to(bp_user[None, :], (8, H))
    w1t = mlp_W1[4].T
    w2t = mlp_W2[4].T
    g1b = jnp.broadcast_to(mlp_g1[4][None, :], (8, H))
    be1b = jnp.broadcast_to(mlp_be1[4][None, :], (8, H))
    g2b = jnp.broadcast_to(mlp_g2[4][None, :], (8, H))
    be2b = jnp.broadcast_to(mlp_be2[4][None, :], (8, H))
    o1t = out_W1.T
    ob1 = jnp.broadcast_to(out_b1[None, :], (8, H))
    o2t = out_W2.T
    ob2 = jnp.broadcast_to(out_b2[None, :], (8, H))

    pad = E2 - E
    src_p = jnp.concatenate([ei_similar[0], jnp.arange(pad, dtype=jnp.int32)])
    dst_p = jnp.concatenate(
        [ei_similar[1], jnp.full((pad,), NPAD, jnp.int32)])
    ed = jnp.concatenate([src_p.reshape(NT, NCH, G),
                          dst_p.reshape(NT, NCH, G)], axis=-1)

    h = _proj(x_user, wpt, bp)

    for layer in range(2):
        aggr = _sc_aggr(h, ed)
        u1, st1 = _p1(one_eps, h, aggr, w1t)
        u2, st2 = _p2(u1, st1, g1b, be1b, w2t)
        h = _combine(pa, h, u2, st2, g2b, be2b, prelu=(layer == 0))

    return _outmlp(h, o1t, ob1, o2t, ob2)


# speedup vs baseline: 2.2952x; 1.0845x over previous
import functools

import jax
import jax.numpy as jnp
from jax import lax
from jax.experimental import pallas as pl
from jax.experimental.pallas import tpu as pltpu
from jax.experimental.pallas import tpu_sc as plsc

N = 50000
H = 128
E = 120000

BM = 5000
GRID = N // BM
NT = 16
E2 = 122880
EPT = E2 // NT
NCH = 120
G = EPT // NCH
NPAD = 50176
RR = NPAD // 4
TRASH = 256
ZC = 28

_f32 = jnp.float32


HCH = NCH // 3


def _sc_aggr_body(h_hbm, ed_hbm, aggr, acc, ed_v, idxm, ga, gb,
                  sga, sgb, ssa, ssb):
    cid = lax.axis_index("c")
    tid = lax.axis_index("s")
    zrow0 = tid * (RR // NT)

    def _wait(sem, buf):
        pltpu.make_async_copy(h_hbm.at[pl.ds(0, G)], buf, sem).wait()

    def _remap(j, b, base):
        @pl.loop(0, G // 16)
        def _g(g):
            d = ed_v[j, pl.ds(G + g * 16, 16)]
            m = (d >= base) & (d < base + RR)
            t = RR + ((j * G + g * 16 + lax.iota(jnp.int32, 16)) & (TRASH - 1))
            idxm[b, pl.ds(g * 16, 16)] = jnp.where(m, d - base, t)

    for p in range(2):
        base = (2 * cid + p) * RR

        @pl.loop(0, ZC)
        def _zfill(r):
            for k in range(8):
                ga[r, pl.ds(k * 16, 16)] = jnp.zeros((16,), _f32)

        @pl.loop(0, RR // NT // ZC)
        def _zero(z):
            pltpu.sync_copy(ga.at[pl.ds(0, ZC)],
                            acc.at[pl.ds(zrow0 + z * ZC, ZC)])

        plsc.subcore_barrier()

        for half in range(3):
            pltpu.sync_copy(ed_hbm.at[tid, pl.ds(half * HCH, HCH)], ed_v)
            pltpu.async_copy(h_hbm.at[ed_v.at[0, pl.ds(0, G)]], ga, sga)

            @pl.loop(0, HCH // 2)
            def _pair(q):
                j0 = 2 * q
                j1 = 2 * q + 1
                _remap(j0, 0, base)

                @pl.when(q > 0)
                def _():
                    _wait(ssb, gb)

                pltpu.async_copy(h_hbm.at[ed_v.at[j1, pl.ds(0, G)]], gb, sgb)
                _wait(sga, ga)
                pltpu.async_copy(ga, acc.at[idxm.at[0]], ssa, add=True)
                _remap(j1, 1, base)
                _wait(ssa, ga)

                @pl.when(q < HCH // 2 - 1)
                def _():
                    pltpu.async_copy(
                        h_hbm.at[ed_v.at[j0 + 2, pl.ds(0, G)]], ga, sga)

                _wait(sgb, gb)
                pltpu.async_copy(gb, acc.at[idxm.at[1]], ssb, add=True)

            _wait(ssb, gb)

        plsc.subcore_barrier()
        pltpu.sync_copy(acc.at[pl.ds(tid * (RR // NT), RR // NT)],
                        aggr.at[pl.ds(base + tid * (RR // NT), RR // NT)])
        plsc.subcore_barrier()


def _sc_aggr(h, ed):
    mesh = plsc.VectorSubcoreMesh(core_axis_name="c", subcore_axis_name="s",
                                  num_cores=2, num_subcores=NT)
    fn = pl.kernel(
        _sc_aggr_body,
        out_type=jax.ShapeDtypeStruct((NPAD, H), _f32),
        mesh=mesh,
        scratch_types=[
            pltpu.VMEM_SHARED((RR + TRASH, H), _f32),
            pltpu.VMEM((HCH, 2 * G), jnp.int32),
            pltpu.VMEM((2, G), jnp.int32),
            pltpu.VMEM((G, H), _f32),
            pltpu.VMEM((G, H), _f32),
            pltpu.SemaphoreType.DMA,
            pltpu.SemaphoreType.DMA,
            pltpu.SemaphoreType.DMA,
            pltpu.SemaphoreType.DMA,
        ],
    )
    return fn(h, ed)


_CP = pltpu.CompilerParams(dimension_semantics=("arbitrary",))


def _proj_body(x_ref, w_ref, b_ref, o_ref):
    o_ref[...] = (jnp.dot(x_ref[...], w_ref[...], preferred_element_type=_f32)
                  + b_ref[0:1, :])


def _proj(x, wpt, bp):
    return pl.pallas_call(
        _proj_body,
        grid=(GRID,),
        in_specs=[
            pl.BlockSpec((BM, H), lambda i: (i, 0)),
            pl.BlockSpec((H, H), lambda i: (0, 0)),
            pl.BlockSpec((8, H), lambda i: (0, 0)),
        ],
        out_specs=pl.BlockSpec((BM, H), lambda i: (i, 0)),
        out_shape=jax.ShapeDtypeStruct((NPAD, H), _f32),
        compiler_params=_CP,
    )(x, wpt, bp)


def _p1_body(e_ref, h_ref, a_ref, w_ref, u_ref, st_ref):
    i = pl.program_id(0)
    comb = e_ref[0, 0] * h_ref[...] + a_ref[...]
    u = jnp.dot(comb, w_ref[...], preferred_element_type=_f32)
    u_ref[...] = u
    s = jnp.sum(u, axis=0, keepdims=True)
    q = jnp.sum(u * u, axis=0, keepdims=True)
    st = jnp.concatenate([s, q, jnp.zeros((6, H), _f32)], axis=0)

    @pl.when(i == 0)
    def _():
        st_ref[...] = st

    @pl.when(i > 0)
    def _():
        st_ref[...] = st_ref[...] + st


def _p1(one_eps, h, aggr, w1t):
    return pl.pallas_call(
        _p1_body,
        grid=(GRID,),
        in_specs=[
            pl.BlockSpec(memory_space=pltpu.SMEM),
            pl.BlockSpec((BM, H), lambda i: (i, 0)),
            pl.BlockSpec((BM, H), lambda i: (i, 0)),
            pl.BlockSpec((H, H), lambda i: (0, 0)),
        ],
        out_specs=[
            pl.BlockSpec((BM, H), lambda i: (i, 0)),
            pl.BlockSpec((8, H), lambda i: (0, 0)),
        ],
        out_shape=[
            jax.ShapeDtypeStruct((N, H), _f32),
            jax.ShapeDtypeStruct((8, H), _f32),
        ],
        compiler_params=_CP,
    )(one_eps, h, aggr, w1t)


def _p2_body(u_ref, st1_ref, g_ref, be_ref, w_ref, o_ref, st_ref):
    i = pl.program_id(0)
    m = st1_ref[0:1, :] * (1.0 / N)
    v = st1_ref[1:2, :] * (1.0 / N) - m * m
    scale = g_ref[0:1, :] * lax.rsqrt(v + 1e-5)
    shift = be_ref[0:1, :] - m * scale
    r = jnp.maximum(u_ref[...] * scale + shift, 0.0)
    u2 = jnp.dot(r, w_ref[...], preferred_element_type=_f32)
    o_ref[...] = u2
    s = jnp.sum(u2, axis=0, keepdims=True)
    q = jnp.sum(u2 * u2, axis=0, keepdims=True)
    st = jnp.concatenate([s, q, jnp.zeros((6, H), _f32)], axis=0)

    @pl.when(i == 0)
    def _():
        st_ref[...] = st

    @pl.when(i > 0)
    def _():
        st_ref[...] = st_ref[...] + st


def _p2(u1, st1, g1b, be1b, w2t):
    return pl.pallas_call(
        _p2_body,
        grid=(GRID,),
        in_specs=[
            pl.BlockSpec((BM, H), lambda i: (i, 0)),
            pl.BlockSpec((8, H), lambda i: (0, 0)),
            pl.BlockSpec((8, H), lambda i: (0, 0)),
            pl.BlockSpec((8, H), lambda i: (0, 0)),
            pl.BlockSpec((H, H), lambda i: (0, 0)),
        ],
        out_specs=[
            pl.BlockSpec((BM, H), lambda i: (i, 0)),
            pl.BlockSpec((8, H), lambda i: (0, 0)),
        ],
        out_shape=[
            jax.ShapeDtypeStruct((N, H), _f32),
            jax.ShapeDtypeStruct((8, H), _f32),
        ],
        compiler_params=_CP,
    )(u1, st1, g1b, be1b, w2t)


def _combine_body(pa_ref, h_ref, v_ref, st2_ref, g_ref, be_ref, o_ref, *,
                  prelu):
    m = st2_ref[0:1, :] * (1.0 / N)
    v = st2_ref[1:2, :] * (1.0 / N) - m * m
    scale = g_ref[0:1, :] * lax.rsqrt(v + 1e-5)
    shift = be_ref[0:1, :] - m * scale
    x = h_ref[...] + v_ref[...] * scale + shift
    if prelu:
        x = jnp.where(x > 0, x, pa_ref[0, 0] * x)
    o_ref[...] = x


def _combine(pa, h, u2, st2, g2b, be2b, prelu):
    return pl.pallas_call(
        functools.partial(_combine_body, prelu=prelu),
        grid=(GRID,),
        in_specs=[pl.BlockSpec(memory_space=pltpu.SMEM)]
        + [pl.BlockSpec((BM, H), lambda i: (i, 0)) for _ in range(2)]
        + [pl.BlockSpec((8, H), lambda i: (0, 0)) for _ in range(3)],
        out_specs=pl.BlockSpec((BM, H), lambda i: (i, 0)),
        out_shape=jax.ShapeDtypeStruct((NPAD, H), _f32),
        compiler_params=_CP,
    )(pa, h, u2, st2, g2b, be2b)


def _out_body(h_ref, w1_ref, b1_ref, w2_ref, b2_ref, o_ref):
    t = jnp.dot(h_ref[...], w1_ref[...], preferred_element_type=_f32)
    t = jnp.maximum(t + b1_ref[0:1, :], 0.0)
    o_ref[...] = jnp.dot(t, w2_ref[...], preferred_element_type=_f32) + b2_ref[0:1, :]


def _outmlp(h, o1t, ob1, o2t, ob2):
    return pl.pallas_call(
        _out_body,
        grid=(GRID,),
        in_specs=[
            pl.BlockSpec((BM, H), lambda i: (i, 0)),
            pl.BlockSpec((H, H), lambda i: (0, 0)),
            pl.BlockSpec((8, H), lambda i: (0, 0)),
            pl.BlockSpec((H, H), lambda i: (0, 0)),
            pl.BlockSpec((8, H), lambda i: (0, 0)),
        ],
        out_specs=pl.BlockSpec((BM, H), lambda i: (i, 0)),
        out_shape=jax.ShapeDtypeStruct((N, H), _f32),
        compiler_params=_CP,
    )(h, o1t, ob1, o2t, ob2)


def kernel(x_user, x_question, x_answer, ei_asks, ei_answers, ei_contains,
           ei_rates, ei_similar, Wp_user, bp_user, Wp_question, bp_question,
           Wp_answer, bp_answer, mlp_W1, mlp_b1, mlp_g1, mlp_be1, mlp_W2,
           mlp_b2, mlp_g2, mlp_be2, epsilon, prelu_a, out_W1, out_b1,
           out_W2, out_b2):
    one_eps = (1.0 + epsilon).astype(_f32).reshape(1, 1)
    pa = prelu_a.astype(_f32).reshape(1, 1)

    wpt = Wp_user.T
    bp = jnp.broadcast_to(bp_user[None, :], (8, H))
    w1t = mlp_W1[4].T
    w2t = mlp_W2[4].T
    g1b = jnp.broadcast_to(mlp_g1[4][None, :], (8, H))
    be1b = jnp.broadcast_to(mlp_be1[4][None, :], (8, H))
    g2b = jnp.broadcast_to(mlp_g2[4][None, :], (8, H))
    be2b = jnp.broadcast_to(mlp_be2[4][None, :], (8, H))
    o1t = out_W1.T
    ob1 = jnp.broadcast_to(out_b1[None, :], (8, H))
    o2t = out_W2.T
    ob2 = jnp.broadcast_to(out_b2[None, :], (8, H))

    pad = E2 - E
    src_p = jnp.concatenate([ei_similar[0], jnp.arange(pad, dtype=jnp.int32)])
    dst_p = jnp.concatenate(
        [ei_similar[1], jnp.full((pad,), NPAD, jnp.int32)])
    ed = jnp.concatenate([src_p.reshape(NT, NCH, G),
                          dst_p.reshape(NT, NCH, G)], axis=-1)

    h = _proj(x_user, wpt, bp)

    for layer in range(2):
        aggr = _sc_aggr(h, ed)
        u1, st1 = _p1(one_eps, h, aggr, w1t)
        u2, st2 = _p2(u1, st1, g1b, be1b, w2t)
        h = _combine(pa, h, u2, st2, g2b, be2b, prelu=(layer == 0))

    return _outmlp(h, o1t, ob1, o2t, ob2)


# speedup vs baseline: 2.3607x; 1.0286x over previous
import functools

import jax
import jax.numpy as jnp
from jax import lax
from jax.experimental import pallas as pl
from jax.experimental.pallas import tpu as pltpu
from jax.experimental.pallas import tpu_sc as plsc

N = 50000
H = 128
E = 120000

BM = 10000
GRID = N // BM
NT = 16
E2 = 122880
EPT = E2 // NT
NCH = 120
G = EPT // NCH
NPAD = 50176
RR = NPAD // 4
TRASH = 256
ZC = 28

_f32 = jnp.float32


HCH = NCH // 3


def _sc_aggr_body(h_hbm, ed_hbm, aggr, acc, ed_v, idxm, ga, gb,
                  sga, sgb, ssa, ssb):
    cid = lax.axis_index("c")
    tid = lax.axis_index("s")
    zrow0 = tid * (RR // NT)

    def _wait(sem, buf):
        pltpu.make_async_copy(h_hbm.at[pl.ds(0, G)], buf, sem).wait()

    def _remap(j, b, base):
        @pl.loop(0, G // 16)
        def _g(g):
            d = ed_v[j, pl.ds(G + g * 16, 16)]
            m = (d >= base) & (d < base + RR)
            t = RR + ((j * G + g * 16 + lax.iota(jnp.int32, 16)) & (TRASH - 1))
            idxm[b, pl.ds(g * 16, 16)] = jnp.where(m, d - base, t)

    for p in range(2):
        base = (2 * cid + p) * RR

        @pl.loop(0, ZC)
        def _zfill(r):
            for k in range(8):
                ga[r, pl.ds(k * 16, 16)] = jnp.zeros((16,), _f32)

        @pl.loop(0, RR // NT // ZC)
        def _zero(z):
            pltpu.sync_copy(ga.at[pl.ds(0, ZC)],
                            acc.at[pl.ds(zrow0 + z * ZC, ZC)])

        plsc.subcore_barrier()

        for half in range(3):
            pltpu.sync_copy(ed_hbm.at[tid, pl.ds(half * HCH, HCH)], ed_v)
            pltpu.async_copy(h_hbm.at[ed_v.at[0, pl.ds(0, G)]], ga, sga)

            @pl.loop(0, HCH // 2)
            def _pair(q):
                j0 = 2 * q
                j1 = 2 * q + 1
                _remap(j0, 0, base)

                @pl.when(q > 0)
                def _():
                    _wait(ssb, gb)

                pltpu.async_copy(h_hbm.at[ed_v.at[j1, pl.ds(0, G)]], gb, sgb)
                _wait(sga, ga)
                pltpu.async_copy(ga, acc.at[idxm.at[0]], ssa, add=True)
                _remap(j1, 1, base)
                _wait(ssa, ga)

                @pl.when(q < HCH // 2 - 1)
                def _():
                    pltpu.async_copy(
                        h_hbm.at[ed_v.at[j0 + 2, pl.ds(0, G)]], ga, sga)

                _wait(sgb, gb)
                pltpu.async_copy(gb, acc.at[idxm.at[1]], ssb, add=True)

            _wait(ssb, gb)

        plsc.subcore_barrier()
        pltpu.sync_copy(acc.at[pl.ds(tid * (RR // NT), RR // NT)],
                        aggr.at[pl.ds(base + tid * (RR // NT), RR // NT)])
        plsc.subcore_barrier()


def _sc_aggr(h, ed):
    mesh = plsc.VectorSubcoreMesh(core_axis_name="c", subcore_axis_name="s",
                                  num_cores=2, num_subcores=NT)
    fn = pl.kernel(
        _sc_aggr_body,
        out_type=jax.ShapeDtypeStruct((NPAD, H), _f32),
        mesh=mesh,
        scratch_types=[
            pltpu.VMEM_SHARED((RR + TRASH, H), _f32),
            pltpu.VMEM((HCH, 2 * G), jnp.int32),
            pltpu.VMEM((2, G), jnp.int32),
            pltpu.VMEM((G, H), _f32),
            pltpu.VMEM((G, H), _f32),
            pltpu.SemaphoreType.DMA,
            pltpu.SemaphoreType.DMA,
            pltpu.SemaphoreType.DMA,
            pltpu.SemaphoreType.DMA,
        ],
    )
    return fn(h, ed)


_CP = pltpu.CompilerParams(dimension_semantics=("arbitrary",))


def _proj_body(x_ref, w_ref, b_ref, o_ref):
    o_ref[...] = (jnp.dot(x_ref[...], w_ref[...], preferred_element_type=_f32)
                  + b_ref[0:1, :])


def _proj(x, wpt, bp):
    return pl.pallas_call(
        _proj_body,
        grid=(GRID,),
        in_specs=[
            pl.BlockSpec((BM, H), lambda i: (i, 0)),
            pl.BlockSpec((H, H), lambda i: (0, 0)),
            pl.BlockSpec((8, H), lambda i: (0, 0)),
        ],
        out_specs=pl.BlockSpec((BM, H), lambda i: (i, 0)),
        out_shape=jax.ShapeDtypeStruct((NPAD, H), _f32),
        compiler_params=_CP,
    )(x, wpt, bp)


def _p1_body(e_ref, h_ref, a_ref, w_ref, u_ref, st_ref):
    i = pl.program_id(0)
    comb = e_ref[0, 0] * h_ref[...] + a_ref[...]
    u = jnp.dot(comb, w_ref[...], preferred_element_type=_f32)
    u_ref[...] = u
    s = jnp.sum(u, axis=0, keepdims=True)
    q = jnp.sum(u * u, axis=0, keepdims=True)
    st = jnp.concatenate([s, q, jnp.zeros((6, H), _f32)], axis=0)

    @pl.when(i == 0)
    def _():
        st_ref[...] = st

    @pl.when(i > 0)
    def _():
        st_ref[...] = st_ref[...] + st


def _p1(one_eps, h, aggr, w1t):
    return pl.pallas_call(
        _p1_body,
        grid=(GRID,),
        in_specs=[
            pl.BlockSpec(memory_space=pltpu.SMEM),
            pl.BlockSpec((BM, H), lambda i: (i, 0)),
            pl.BlockSpec((BM, H), lambda i: (i, 0)),
            pl.BlockSpec((H, H), lambda i: (0, 0)),
        ],
        out_specs=[
            pl.BlockSpec((BM, H), lambda i: (i, 0)),
            pl.BlockSpec((8, H), lambda i: (0, 0)),
        ],
        out_shape=[
            jax.ShapeDtypeStruct((N, H), _f32),
            jax.ShapeDtypeStruct((8, H), _f32),
        ],
        compiler_params=_CP,
    )(one_eps, h, aggr, w1t)


def _p2_body(u_ref, st1_ref, g_ref, be_ref, w_ref, o_ref, st_ref):
    i = pl.program_id(0)
    m = st1_ref[0:1, :] * (1.0 / N)
    v = st1_ref[1:2, :] * (1.0 / N) - m * m
    scale = g_ref[0:1, :] * lax.rsqrt(v + 1e-5)
    shift = be_ref[0:1, :] - m * scale
    r = jnp.maximum(u_ref[...] * scale + shift, 0.0)
    u2 = jnp.dot(r, w_ref[...], preferred_element_type=_f32)
    o_ref[...] = u2
    s = jnp.sum(u2, axis=0, keepdims=True)
    q = jnp.sum(u2 * u2, axis=0, keepdims=True)
    st = jnp.concatenate([s, q, jnp.zeros((6, H), _f32)], axis=0)

    @pl.when(i == 0)
    def _():
        st_ref[...] = st

    @pl.when(i > 0)
    def _():
        st_ref[...] = st_ref[...] + st


def _p2(u1, st1, g1b, be1b, w2t):
    return pl.pallas_call(
        _p2_body,
        grid=(GRID,),
        in_specs=[
            pl.BlockSpec((BM, H), lambda i: (i, 0)),
            pl.BlockSpec((8, H), lambda i: (0, 0)),
            pl.BlockSpec((8, H), lambda i: (0, 0)),
            pl.BlockSpec((8, H), lambda i: (0, 0)),
            pl.BlockSpec((H, H), lambda i: (0, 0)),
        ],
        out_specs=[
            pl.BlockSpec((BM, H), lambda i: (i, 0)),
            pl.BlockSpec((8, H), lambda i: (0, 0)),
        ],
        out_shape=[
            jax.ShapeDtypeStruct((N, H), _f32),
            jax.ShapeDtypeStruct((8, H), _f32),
        ],
        compiler_params=_CP,
    )(u1, st1, g1b, be1b, w2t)


def _combine_body(pa_ref, h_ref, v_ref, st2_ref, g_ref, be_ref, o_ref, *,
                  prelu):
    m = st2_ref[0:1, :] * (1.0 / N)
    v = st2_ref[1:2, :] * (1.0 / N) - m * m
    scale = g_ref[0:1, :] * lax.rsqrt(v + 1e-5)
    shift = be_ref[0:1, :] - m * scale
    x = h_ref[...] + v_ref[...] * scale + shift
    if prelu:
        x = jnp.where(x > 0, x, pa_ref[0, 0] * x)
    o_ref[...] = x


def _combine(pa, h, u2, st2, g2b, be2b, prelu):
    return pl.pallas_call(
        functools.partial(_combine_body, prelu=prelu),
        grid=(GRID,),
        in_specs=[pl.BlockSpec(memory_space=pltpu.SMEM)]
        + [pl.BlockSpec((BM, H), lambda i: (i, 0)) for _ in range(2)]
        + [pl.BlockSpec((8, H), lambda i: (0, 0)) for _ in range(3)],
        out_specs=pl.BlockSpec((BM, H), lambda i: (i, 0)),
        out_shape=jax.ShapeDtypeStruct((NPAD, H), _f32),
        compiler_params=_CP,
    )(pa, h, u2, st2, g2b, be2b)


def _out_body(h_ref, w1_ref, b1_ref, w2_ref, b2_ref, o_ref):
    t = jnp.dot(h_ref[...], w1_ref[...], preferred_element_type=_f32)
    t = jnp.maximum(t + b1_ref[0:1, :], 0.0)
    o_ref[...] = jnp.dot(t, w2_ref[...], preferred_element_type=_f32) + b2_ref[0:1, :]


def _outmlp(h, o1t, ob1, o2t, ob2):
    return pl.pallas_call(
        _out_body,
        grid=(GRID,),
        in_specs=[
            pl.BlockSpec((BM, H), lambda i: (i, 0)),
            pl.BlockSpec((H, H), lambda i: (0, 0)),
            pl.BlockSpec((8, H), lambda i: (0, 0)),
            pl.BlockSpec((H, H), lambda i: (0, 0)),
            pl.BlockSpec((8, H), lambda i: (0, 0)),
        ],
        out_specs=pl.BlockSpec((BM, H), lambda i: (i, 0)),
        out_shape=jax.ShapeDtypeStruct((N, H), _f32),
        compiler_params=_CP,
    )(h, o1t, ob1, o2t, ob2)


def kernel(x_user, x_question, x_answer, ei_asks, ei_answers, ei_contains,
           ei_rates, ei_similar, Wp_user, bp_user, Wp_question, bp_question,
           Wp_answer, bp_answer, mlp_W1, mlp_b1, mlp_g1, mlp_be1, mlp_W2,
           mlp_b2, mlp_g2, mlp_be2, epsilon, prelu_a, out_W1, out_b1,
           out_W2, out_b2):
    one_eps = (1.0 + epsilon).astype(_f32).reshape(1, 1)
    pa = prelu_a.astype(_f32).reshape(1, 1)

    wpt = Wp_user.T
    bp = jnp.broadcast_to(bp_user[None, :], (8, H))
    w1t = mlp_W1[4].T
    w2t = mlp_W2[4].T
    g1b = jnp.broadcast_to(mlp_g1[4][None, :], (8, H))
    be1b = jnp.broadcast_to(mlp_be1[4][None, :], (8, H))
    g2b = jnp.broadcast_to(mlp_g2[4][None, :], (8, H))
    be2b = jnp.broadcast_to(mlp_be2[4][None, :], (8, H))
    o1t = out_W1.T
    ob1 = jnp.broadcast_to(out_b1[None, :], (8, H))
    o2t = out_W2.T
    ob2 = jnp.broadcast_to(out_b2[None, :], (8, H))

    pad = E2 - E
    src_p = jnp.concatenate([ei_similar[0], jnp.arange(pad, dtype=jnp.int32)])
    dst_p = jnp.concatenate(
        [ei_similar[1], jnp.full((pad,), NPAD, jnp.int32)])
    ed = jnp.concatenate([src_p.reshape(NT, NCH, G),
                          dst_p.reshape(NT, NCH, G)], axis=-1)

    h = _proj(x_user, wpt, bp)

    for layer in range(2):
        aggr = _sc_aggr(h, ed)
        u1, st1 = _p1(one_eps, h, aggr, w1t)
        u2, st2 = _p2(u1, st1, g1b, be1b, w2t)
        h = _combine(pa, h, u2, st2, g2b, be2b, prelu=(layer == 0))

    return _outmlp(h, o1t, ob1, o2t, ob2)
